# baseline (speedup 1.0000x reference)
import functools
import math

import jax
import jax.numpy as jnp
from jax import lax
from jax.experimental import pallas as pl
from jax.experimental.pallas import tpu as pltpu
from jax.experimental.pallas import tpu_sc as plsc

_L_MAX = 2
_NR = 4
_KERNEL_RADIUS = 2.0
_PATCH_K = 64
_SIGMA = _KERNEL_RADIUS / (_NR - 1)
_INV_2SIG2 = 1.0 / (2.0 * _SIGMA * _SIGMA)

_C0 = 0.5 * math.sqrt(1.0 / math.pi)
_C1 = math.sqrt(3.0 / (4.0 * math.pi))
_C2A = 0.5 * math.sqrt(15.0 / math.pi)
_C2B = 0.25 * math.sqrt(5.0 / math.pi)
_C2C = 0.25 * math.sqrt(15.0 / math.pi)

_VB = 128
_NB = 512

_NUM_WORKERS = 32


def _rb(t):
    return t.astype(jnp.bfloat16).astype(jnp.float32)


def _prep_body(xn_ref, out_ref):
    xn = xn_ref[0]
    x, y, z = xn[0:1, :], xn[1:2, :], xn[2:3, :]
    nn = x * x + y * y + z * z
    zero = jnp.zeros_like(x)
    out_ref[0] = jnp.concatenate(
        [_rb(x), _rb(y), _rb(z), nn, x, y, z, zero], axis=0)


def _make_sc_select(B, N):
    sc_r = (B * N) // _NUM_WORKERS
    w_per_b = N // sc_r
    mesh = plsc.VectorSubcoreMesh(core_axis_name="c", subcore_axis_name="s")

    @functools.partial(
        pl.kernel, mesh=mesh,
        out_type=jax.ShapeDtypeStruct((B * N,), jnp.int32),
        scratch_types=[
            pltpu.VMEM((8, N), jnp.float32),
            pltpu.VMEM((N,), jnp.int32),
            pltpu.VMEM((256,), jnp.int32),
            pltpu.VMEM((sc_r,), jnp.int32),
        ],
        compiler_params=pltpu.CompilerParams(needs_layout_passes=False),
    )
    def sc_select(xdat_hbm, pout_hbm, xd_v, ub_v, hist_v, pbuf_v):
        cid = lax.axis_index("c")
        sid = lax.axis_index("s")
        wid = sid * 2 + cid
        b = wid // w_per_b
        v0 = (wid % w_per_b) * sc_r
        pltpu.sync_copy(xdat_hbm.at[b], xd_v)
        lanes = lax.iota(jnp.int32, 16)
        ones = jnp.ones((16,), jnp.int32)
        zeros16 = jnp.zeros((16,), jnp.int32)

        def scan_hist(nvreg, k):
            def s(t, car):
                tot, bidx, cbel = car
                cum = plsc.cumsum(hist_v[pl.ds(t * 16, 16)]) + tot
                ltk = cum < k
                nlt = jnp.max(plsc.all_reduce_population_count(ltk))
                mx = jnp.max(jnp.where(ltk, cum, 0))
                return (jnp.max(cum), bidx + nlt, jnp.maximum(cbel, mx))
            _, bidx, cbel = lax.fori_loop(
                0, nvreg, s, (jnp.int32(0), jnp.int32(0), jnp.int32(0)))
            return bidx, cbel

        def row_body(r, carry):
            vg = v0 + r
            idxr = zeros16 + vg
            vxb = plsc.load_gather(xd_v, [zeros16, idxr])
            vyb = plsc.load_gather(xd_v, [zeros16 + 1, idxr])
            vzb = plsc.load_gather(xd_v, [zeros16 + 2, idxr])
            nv = plsc.load_gather(xd_v, [zeros16 + 3, idxr])
            for t in range(16):
                hist_v[pl.ds(t * 16, 16)] = zeros16

            def p1(j, _):
                sl = pl.ds(j * 16, 16)
                m = vxb * xd_v[0, sl] + vyb * xd_v[1, sl] + vzb * xd_v[2, sl]
                Dv = nv - 2.0 * m + xd_v[3, sl]
                u = plsc.bitcast(jnp.maximum(Dv, 0.0), jnp.int32)
                ub_v[sl] = u
                plsc.addupdate_scatter(
                    hist_v, [lax.shift_right_logical(u, 23)], ones)
                return 0
            lax.fori_loop(0, N // 16, p1, 0)
            b1, c1 = scan_hist(16, jnp.int32(_PATCH_K))

            def lvl(shift_pref, pref, sh, nbits, k):
                nb = 1 << nbits
                for t in range(nb // 16):
                    hist_v[pl.ds(t * 16, 16)] = zeros16
                msk = jnp.int32(nb - 1)

                def pj(j, _):
                    sl = pl.ds(j * 16, 16)
                    u = ub_v[sl]
                    pm = lax.shift_right_logical(u, shift_pref) == pref
                    bk = lax.shift_right_logical(u, sh) & msk
                    plsc.addupdate_scatter(hist_v, [bk], ones, mask=pm)
                    return 0
                lax.fori_loop(0, N // 16, pj, 0)
                return scan_hist(nb // 16, k)

            k2 = _PATCH_K - c1
            b2, c2 = lvl(23, b1, 15, 8, k2)
            pref2 = (b1 << 8) | b2
            k3 = k2 - c2
            b3, c3 = lvl(15, pref2, 7, 8, k3)
            pref3 = (pref2 << 8) | b3
            k4 = k3 - c3
            b4, _ = lvl(7, pref3, 0, 7, k4)
            P = (pref3 << 7) | b4
            plsc.store_scatter(pbuf_v, [zeros16 + r], zeros16 + P,
                               mask=lanes == 0)
            return carry

        lax.fori_loop(0, sc_r, row_body, 0)
        pltpu.sync_copy(pbuf_v, pout_hbm.at[pl.ds(wid * sc_r, sc_r)])

    return sc_select


def _tc_body(xv_ref, xn_ref, sig_ref, wf_ref, b_ref, out_ref, d_scr):
    N = xn_ref.shape[2]
    C = sig_ref.shape[2]
    xv = xv_ref[0]
    xn = xn_ref[0]
    sig = sig_ref[0]

    vx, vy, vz = xv[:, 0:1], xv[:, 1:2], xv[:, 2:3]
    P = jax.lax.bitcast_convert_type(xv[:, 3:4], jnp.int32)
    nx, ny, nz = xn[0:1, :], xn[1:2, :], xn[2:3, :]
    nv = vx * vx + vy * vy + vz * vz
    nn = nx * nx + ny * ny + nz * nz

    m = _rb(vx) * _rb(nx) + _rb(vy) * _rb(ny) + _rb(vz) * _rb(nz)
    d_scr[...] = nv - 2.0 * m + nn
    D = d_scr[...]

    ub = jax.lax.bitcast_convert_type(jnp.maximum(D, 0.0), jnp.int32)
    mask = (ub <= P).astype(jnp.float32)

    dist = jnp.sqrt(jnp.maximum(D, 1e-4))
    rad0 = jnp.exp(-(dist * dist) * _INV_2SIG2)
    y_w = jnp.sum(mask * rad0, axis=1, keepdims=True) * _C0
    g = mask * (1.0 / (y_w + 1e-6))

    VB = xv.shape[0]
    acc = jnp.zeros((_NR * 9 * VB, C), jnp.float32)
    for c in range(N // _NB):
        sl = slice(c * _NB, (c + 1) * _NB)
        distc = dist[:, sl]
        gc = g[:, sl]
        dx = nx[:, sl] - vx
        dy = ny[:, sl] - vy
        dz = nz[:, sl] - vz
        inv = jax.lax.rsqrt(dx * dx + dy * dy + dz * dz + 1e-8)
        ux, uy, uz = dx * inv, dy * inv, dz * inv
        Ys = (jnp.full_like(ux, _C0), _C1 * uy, _C1 * uz, _C1 * ux,
              _C2A * ux * uy, _C2A * uy * uz, _C2B * (3.0 * uz * uz - 1.0),
              _C2A * ux * uz, _C2C * (ux * ux - uy * uy))
        slabs = []
        for r in range(_NR):
            rr = distc - r * (_KERNEL_RADIUS / (_NR - 1))
            grc = gc * jnp.exp(-(rr * rr) * _INV_2SIG2)
            for s in range(9):
                slabs.append(grc * Ys[s])
        A = jnp.concatenate(slabs, axis=0)
        acc = acc + jnp.dot(A.astype(jnp.bfloat16),
                            sig[sl, :].astype(jnp.bfloat16),
                            preferred_element_type=jnp.float32)

    sq = acc * acc
    pieces = []
    for r in range(_NR):
        base = r * 9 * VB
        p0 = sq[base:base + VB]
        p1 = (sq[base + VB:base + 2 * VB]
              + sq[base + 2 * VB:base + 3 * VB]
              + sq[base + 3 * VB:base + 4 * VB])
        p2 = (sq[base + 4 * VB:base + 5 * VB]
              + sq[base + 5 * VB:base + 6 * VB]
              + sq[base + 6 * VB:base + 7 * VB]
              + sq[base + 7 * VB:base + 8 * VB]
              + sq[base + 8 * VB:base + 9 * VB])
        pieces += [p0, p1, p2]
    cat = jnp.concatenate(pieces, axis=1)
    cat = jnp.sqrt(jnp.maximum(cat, 1e-4))
    out = jnp.dot(cat.astype(jnp.bfloat16),
                  wf_ref[...].astype(jnp.bfloat16),
                  preferred_element_type=jnp.float32) + b_ref[...]
    out_ref[0] = out


def kernel(xyz, signal, weight, biases):
    B, N, _ = xyz.shape
    C = signal.shape[2]
    CO = weight.shape[0]

    xt = jnp.swapaxes(xyz, 1, 2)
    pad5 = jnp.zeros((B, 5, N), jnp.float32)
    xn = jnp.concatenate([xt, pad5], axis=1)

    scdat = pl.pallas_call(
        _prep_body,
        grid=(B,),
        in_specs=[pl.BlockSpec((1, 8, N), lambda b: (b, 0, 0))],
        out_specs=pl.BlockSpec((1, 8, N), lambda b: (b, 0, 0)),
        out_shape=jax.ShapeDtypeStruct((B, 8, N), jnp.float32),
    )(xn)

    P = _make_sc_select(B, N)(scdat)
    pf = jax.lax.bitcast_convert_type(P, jnp.float32).reshape(B, N, 1)
    pad4 = jnp.zeros((B, N, 4), jnp.float32)
    xv = jnp.concatenate([xyz, pf, pad4], axis=2)

    wf = jnp.transpose(weight, (2, 3, 1, 0)).reshape(_NR * 3 * C, CO)
    b2 = biases.reshape(1, CO)

    return pl.pallas_call(
        _tc_body,
        grid=(B, N // _VB),
        in_specs=[
            pl.BlockSpec((1, _VB, 8), lambda b, v: (b, v, 0)),
            pl.BlockSpec((1, 8, N), lambda b, v: (b, 0, 0)),
            pl.BlockSpec((1, N, C), lambda b, v: (b, 0, 0)),
            pl.BlockSpec((_NR * 3 * C, CO), lambda b, v: (0, 0)),
            pl.BlockSpec((1, CO), lambda b, v: (0, 0)),
        ],
        out_specs=pl.BlockSpec((1, _VB, CO), lambda b, v: (b, v, 0)),
        out_shape=jax.ShapeDtypeStruct((B, N, CO), jnp.float32),
        scratch_shapes=[pltpu.VMEM((_VB, N), jnp.float32)],
        compiler_params=pltpu.CompilerParams(
            dimension_semantics=("parallel", "parallel")),
    )(xv, xn, signal.astype(jnp.float32), wf, b2)


# speedup vs baseline: 1.0496x; 1.0496x over previous
import functools
import math

import jax
import jax.numpy as jnp
from jax import lax
from jax.experimental import pallas as pl
from jax.experimental.pallas import tpu as pltpu
from jax.experimental.pallas import tpu_sc as plsc

_L_MAX = 2
_NR = 4
_KERNEL_RADIUS = 2.0
_PATCH_K = 64
_SIGMA = _KERNEL_RADIUS / (_NR - 1)
_INV_2SIG2 = 1.0 / (2.0 * _SIGMA * _SIGMA)

_C0 = 0.5 * math.sqrt(1.0 / math.pi)
_C1 = math.sqrt(3.0 / (4.0 * math.pi))
_C2A = 0.5 * math.sqrt(15.0 / math.pi)
_C2B = 0.25 * math.sqrt(5.0 / math.pi)
_C2C = 0.25 * math.sqrt(15.0 / math.pi)

_VB = 128
_NB = 512

_NUM_WORKERS = 32


def _rb(t):
    return t.astype(jnp.bfloat16).astype(jnp.float32)


def _prep_body(xn_ref, out_ref):
    xn = xn_ref[0]
    x, y, z = xn[0:1, :], xn[1:2, :], xn[2:3, :]
    nn = x * x + y * y + z * z
    zero = jnp.zeros_like(x)
    out_ref[0] = jnp.concatenate(
        [_rb(x), _rb(y), _rb(z), nn, x, y, z, zero], axis=0)


def _make_sc_select(B, N):
    sc_r = (B * N) // _NUM_WORKERS
    w_per_b = N // sc_r
    mesh = plsc.VectorSubcoreMesh(core_axis_name="c", subcore_axis_name="s")

    @functools.partial(
        pl.kernel, mesh=mesh,
        out_type=jax.ShapeDtypeStruct((B * N,), jnp.int32),
        scratch_types=[
            pltpu.VMEM((8, N), jnp.float32),
            pltpu.VMEM((N,), jnp.int32),
            pltpu.VMEM((256,), jnp.int32),
            pltpu.VMEM((sc_r,), jnp.int32),
        ],
        compiler_params=pltpu.CompilerParams(needs_layout_passes=False),
    )
    def sc_select(xdat_hbm, pout_hbm, xd_v, ub_v, hist_v, pbuf_v):
        cid = lax.axis_index("c")
        sid = lax.axis_index("s")
        wid = sid * 2 + cid
        b = wid // w_per_b
        v0 = (wid % w_per_b) * sc_r
        pltpu.sync_copy(xdat_hbm.at[b], xd_v)
        lanes = lax.iota(jnp.int32, 16)
        ones = jnp.ones((16,), jnp.int32)
        zeros16 = jnp.zeros((16,), jnp.int32)

        def scan_hist(nvreg, k):
            def s(t, car):
                tot, bidx, cbel = car
                cum = plsc.cumsum(hist_v[pl.ds(t * 16, 16)]) + tot
                ltk = cum < k
                nlt = jnp.max(plsc.all_reduce_population_count(ltk))
                mx = jnp.max(jnp.where(ltk, cum, 0))
                return (jnp.max(cum), bidx + nlt, jnp.maximum(cbel, mx))
            _, bidx, cbel = lax.fori_loop(
                0, nvreg, s, (jnp.int32(0), jnp.int32(0), jnp.int32(0)))
            return bidx, cbel

        def row_body(r, carry):
            vg = v0 + r
            idxr = zeros16 + vg
            vxb = plsc.load_gather(xd_v, [zeros16, idxr])
            vyb = plsc.load_gather(xd_v, [zeros16 + 1, idxr])
            vzb = plsc.load_gather(xd_v, [zeros16 + 2, idxr])
            nv = plsc.load_gather(xd_v, [zeros16 + 3, idxr])
            for t in range(16):
                hist_v[pl.ds(t * 16, 16)] = zeros16

            def p1(j, _):
                for t in range(8):
                    sl = pl.ds(j * 128 + t * 16, 16)
                    m = (vxb * xd_v[0, sl] + vyb * xd_v[1, sl]
                         + vzb * xd_v[2, sl])
                    Dv = nv - 2.0 * m + xd_v[3, sl]
                    u = plsc.bitcast(jnp.maximum(Dv, 0.0), jnp.int32)
                    ub_v[sl] = u
                    plsc.addupdate_scatter(
                        hist_v, [lax.shift_right_logical(u, 23)], ones)
                return 0
            lax.fori_loop(0, N // 128, p1, 0)
            b1, c1 = scan_hist(16, jnp.int32(_PATCH_K))

            def lvl(shift_pref, pref, sh, nbits, k):
                nb = 1 << nbits
                for t in range(nb // 16):
                    hist_v[pl.ds(t * 16, 16)] = zeros16
                msk = jnp.int32(nb - 1)

                def pj(j, _):
                    for t in range(8):
                        sl = pl.ds(j * 128 + t * 16, 16)
                        u = ub_v[sl]
                        pm = lax.shift_right_logical(u, shift_pref) == pref
                        bk = lax.shift_right_logical(u, sh) & msk
                        plsc.addupdate_scatter(hist_v, [bk], ones, mask=pm)
                    return 0
                lax.fori_loop(0, N // 128, pj, 0)
                return scan_hist(nb // 16, k)

            k2 = _PATCH_K - c1
            b2, c2 = lvl(23, b1, 15, 8, k2)
            pref2 = (b1 << 8) | b2
            k3 = k2 - c2
            b3, c3 = lvl(15, pref2, 7, 8, k3)
            pref3 = (pref2 << 8) | b3
            k4 = k3 - c3
            b4, _ = lvl(7, pref3, 0, 7, k4)
            P = (pref3 << 7) | b4
            plsc.store_scatter(pbuf_v, [zeros16 + r], zeros16 + P,
                               mask=lanes == 0)
            return carry

        lax.fori_loop(0, sc_r, row_body, 0)
        pltpu.sync_copy(pbuf_v, pout_hbm.at[pl.ds(wid * sc_r, sc_r)])

    return sc_select


def _tc_body(xv_ref, xn_ref, sig_ref, wf_ref, b_ref, out_ref, d_scr):
    N = xn_ref.shape[2]
    C = sig_ref.shape[2]
    xv = xv_ref[0]
    xn = xn_ref[0]
    sig = sig_ref[0]

    vx, vy, vz = xv[:, 0:1], xv[:, 1:2], xv[:, 2:3]
    P = jax.lax.bitcast_convert_type(xv[:, 3:4], jnp.int32)
    nx, ny, nz = xn[0:1, :], xn[1:2, :], xn[2:3, :]
    nv = vx * vx + vy * vy + vz * vz
    nn = nx * nx + ny * ny + nz * nz

    m = _rb(vx) * _rb(nx) + _rb(vy) * _rb(ny) + _rb(vz) * _rb(nz)
    d_scr[...] = nv - 2.0 * m + nn
    D = d_scr[...]

    ub = jax.lax.bitcast_convert_type(jnp.maximum(D, 0.0), jnp.int32)
    mask = (ub <= P).astype(jnp.float32)

    dist = jnp.sqrt(jnp.maximum(D, 1e-4))
    rad0 = jnp.exp(-(dist * dist) * _INV_2SIG2)
    y_w = jnp.sum(mask * rad0, axis=1, keepdims=True) * _C0
    g = mask * (1.0 / (y_w + 1e-6))

    VB = xv.shape[0]
    acc = jnp.zeros((_NR * 9 * VB, C), jnp.float32)
    for c in range(N // _NB):
        sl = slice(c * _NB, (c + 1) * _NB)
        distc = dist[:, sl]
        gc = g[:, sl]
        dx = nx[:, sl] - vx
        dy = ny[:, sl] - vy
        dz = nz[:, sl] - vz
        inv = jax.lax.rsqrt(dx * dx + dy * dy + dz * dz + 1e-8)
        ux, uy, uz = dx * inv, dy * inv, dz * inv
        Ys = (jnp.full_like(ux, _C0), _C1 * uy, _C1 * uz, _C1 * ux,
              _C2A * ux * uy, _C2A * uy * uz, _C2B * (3.0 * uz * uz - 1.0),
              _C2A * ux * uz, _C2C * (ux * ux - uy * uy))
        slabs = []
        for r in range(_NR):
            rr = distc - r * (_KERNEL_RADIUS / (_NR - 1))
            grc = gc * jnp.exp(-(rr * rr) * _INV_2SIG2)
            for s in range(9):
                slabs.append(grc * Ys[s])
        A = jnp.concatenate(slabs, axis=0)
        acc = acc + jnp.dot(A.astype(jnp.bfloat16),
                            sig[sl, :].astype(jnp.bfloat16),
                            preferred_element_type=jnp.float32)

    sq = acc * acc
    pieces = []
    for r in range(_NR):
        base = r * 9 * VB
        p0 = sq[base:base + VB]
        p1 = (sq[base + VB:base + 2 * VB]
              + sq[base + 2 * VB:base + 3 * VB]
              + sq[base + 3 * VB:base + 4 * VB])
        p2 = (sq[base + 4 * VB:base + 5 * VB]
              + sq[base + 5 * VB:base + 6 * VB]
              + sq[base + 6 * VB:base + 7 * VB]
              + sq[base + 7 * VB:base + 8 * VB]
              + sq[base + 8 * VB:base + 9 * VB])
        pieces += [p0, p1, p2]
    cat = jnp.concatenate(pieces, axis=1)
    cat = jnp.sqrt(jnp.maximum(cat, 1e-4))
    out = jnp.dot(cat.astype(jnp.bfloat16),
                  wf_ref[...].astype(jnp.bfloat16),
                  preferred_element_type=jnp.float32) + b_ref[...]
    out_ref[0] = out


def kernel(xyz, signal, weight, biases):
    B, N, _ = xyz.shape
    C = signal.shape[2]
    CO = weight.shape[0]

    xt = jnp.swapaxes(xyz, 1, 2)
    pad5 = jnp.zeros((B, 5, N), jnp.float32)
    xn = jnp.concatenate([xt, pad5], axis=1)

    scdat = pl.pallas_call(
        _prep_body,
        grid=(B,),
        in_specs=[pl.BlockSpec((1, 8, N), lambda b: (b, 0, 0))],
        out_specs=pl.BlockSpec((1, 8, N), lambda b: (b, 0, 0)),
        out_shape=jax.ShapeDtypeStruct((B, 8, N), jnp.float32),
    )(xn)

    P = _make_sc_select(B, N)(scdat)
    pf = jax.lax.bitcast_convert_type(P, jnp.float32).reshape(B, N, 1)
    pad4 = jnp.zeros((B, N, 4), jnp.float32)
    xv = jnp.concatenate([xyz, pf, pad4], axis=2)

    wf = jnp.transpose(weight, (2, 3, 1, 0)).reshape(_NR * 3 * C, CO)
    b2 = biases.reshape(1, CO)

    return pl.pallas_call(
        _tc_body,
        grid=(B, N // _VB),
        in_specs=[
            pl.BlockSpec((1, _VB, 8), lambda b, v: (b, v, 0)),
            pl.BlockSpec((1, 8, N), lambda b, v: (b, 0, 0)),
            pl.BlockSpec((1, N, C), lambda b, v: (b, 0, 0)),
            pl.BlockSpec((_NR * 3 * C, CO), lambda b, v: (0, 0)),
            pl.BlockSpec((1, CO), lambda b, v: (0, 0)),
        ],
        out_specs=pl.BlockSpec((1, _VB, CO), lambda b, v: (b, v, 0)),
        out_shape=jax.ShapeDtypeStruct((B, N, CO), jnp.float32),
        scratch_shapes=[pltpu.VMEM((_VB, N), jnp.float32)],
        compiler_params=pltpu.CompilerParams(
            dimension_semantics=("parallel", "parallel")),
    )(xv, xn, signal.astype(jnp.float32), wf, b2)
